# baseline (speedup 1.0000x reference)
import functools

import jax
import jax.numpy as jnp
from jax import lax
from jax.experimental import pallas as pl
from jax.experimental.pallas import tpu as pltpu, tpu_sc as plsc

ENV = 4
F = 64
B = 16384

NC, NS, L = 2, 16, 16
NW = NC * NS
RPW = B // NW
NG = RPW // L
IDXC = 128
NIDX = RPW // IDXC
NT = F // L

_f32 = jnp.float32
_i32 = jnp.int32


def _lane_bcast(v, k):
    idx = jnp.full((L, 1), k, _i32)
    dn = lax.GatherDimensionNumbers(
        offset_dims=(), collapsed_slice_dims=(0,), start_index_map=(0,))
    return lax.gather(v, idx, dn, (1,),
                      mode=lax.GatherScatterMode.PROMISE_IN_BOUNDS)


def _sigmoid(x):
    return 1.0 / (1.0 + jnp.exp(-x))


def _log1to4(s):
    w = (s - 1.0) / (s + 1.0)
    w2 = w * w
    ln = 2.0 * w * (1.0 + w2 * (1.0 / 3.0 + w2 * (0.2 + w2 * (1.0 / 7.0))))
    return ln + s * jnp.exp(-ln) - 1.0


_mesh = plsc.VectorSubcoreMesh(core_axis_name="c", subcore_axis_name="s")


@functools.partial(
    pl.kernel,
    mesh=_mesh,
    compiler_params=pltpu.CompilerParams(
        needs_layout_passes=False, use_tc_tiling_on_sc=False),
    out_type=(
        jax.ShapeDtypeStruct((B,), _f32),
        jax.ShapeDtypeStruct((B,), _f32),
        jax.ShapeDtypeStruct((B * ENV,), _f32),
    ),
    scratch_types=[
        pltpu.VMEM((NIDX, IDXC), _i32),
        pltpu.VMEM((NIDX, IDXC), _i32),
        pltpu.VMEM((NIDX, IDXC), _i32),
        pltpu.VMEM((RPW, F), _f32),
        pltpu.VMEM((RPW, F), _f32),
        pltpu.VMEM((RPW, F), _f32),
        pltpu.VMEM((ENV, F), _f32),
        pltpu.VMEM((L,), _f32),
        pltpu.VMEM((RPW,), _f32),
        pltpu.VMEM((RPW,), _f32),
        pltpu.VMEM((RPW * ENV,), _f32),
        pltpu.SemaphoreType.DMA,
    ],
)
def _sc_forward(u2d, i2d, e2d, wui, wii, wue, wie, wenv_h, clfw_h, clfb_h,
                o_inv, o_env, o_cls,
                idxu_v, idxi_v, idxe_v, rows_u, rows_i, rows_e, clfw_v,
                clfb_v, invs_v, envsc_v, envout_v, sem):
    cid = lax.axis_index("c")
    sid = lax.axis_index("s")
    wid = sid * NC + cid
    base = wid * RPW
    brow = wid * NIDX

    pltpu.sync_copy(u2d.at[pl.ds(brow, NIDX)], idxu_v)
    pltpu.sync_copy(i2d.at[pl.ds(brow, NIDX)], idxi_v)
    pltpu.sync_copy(e2d.at[pl.ds(brow, NIDX)], idxe_v)
    pltpu.sync_copy(clfw_h, clfw_v)
    pltpu.sync_copy(clfb_h, clfb_v)

    def gather_rows(tab, idx_v, dst):
        cps = []
        for j in range(NIDX):
            cps.append(pltpu.async_copy(
                tab.at[idx_v.at[j]], dst.at[pl.ds(j * IDXC, IDXC)], sem))
        return cps

    cps = (gather_rows(wui, idxu_v, rows_u)
           + gather_rows(wii, idxi_v, rows_i)
           + gather_rows(wenv_h, idxe_v, rows_e))
    for cp in cps:
        cp.wait()

    iota = lax.iota(_i32, L)
    masks = [iota == r for r in range(L)]
    bvec = clfb_v[...]
    w = [[clfw_v[k, pl.ds(t * L, L)] for t in range(NT)] for k in range(ENV)]

    def lane_sum_into(acc, vec, r):
        tot = _lane_bcast(plsc.cumsum(vec), L - 1)
        return jnp.where(masks[r], tot, acc)

    def group1(g, _):
        z = jnp.zeros((L,), _f32)
        a0, a1, a2, a3, a4 = z, z, z, z, z
        for r in range(L):
            row = g * L + r
            pt = [rows_u[row, pl.ds(t * L, L)] * rows_i[row, pl.ds(t * L, L)]
                  for t in range(NT)]
            s = (pt[0] + pt[1]) + (pt[2] + pt[3])
            a0 = lane_sum_into(a0, s, r)
            q = [(pt[0] * w[k][0] + pt[1] * w[k][1])
                 + (pt[2] * w[k][2] + pt[3] * w[k][3]) for k in range(ENV)]
            a1 = lane_sum_into(a1, q[0], r)
            a2 = lane_sum_into(a2, q[1], r)
            a3 = lane_sum_into(a3, q[2], r)
            a4 = lane_sum_into(a4, q[3], r)

        invs_v[pl.ds(g * L, L)] = _sigmoid(a0)

        l0 = a1 + _lane_bcast(bvec, 0)
        l1 = a2 + _lane_bcast(bvec, 1)
        l2 = a3 + _lane_bcast(bvec, 2)
        l3 = a4 + _lane_bcast(bvec, 3)
        m = jnp.maximum(jnp.maximum(l0, l1), jnp.maximum(l2, l3))
        e0 = jnp.exp(l0 - m)
        e1 = jnp.exp(l1 - m)
        e2 = jnp.exp(l2 - m)
        e3 = jnp.exp(l3 - m)
        ssum = (e0 + e1) + (e2 + e3)
        lse = m + _log1to4(ssum)
        rl4 = (g * L + iota) * ENV
        plsc.store_scatter(envout_v, [rl4], l0 - lse)
        plsc.store_scatter(envout_v, [rl4 + 1], l1 - lse)
        plsc.store_scatter(envout_v, [rl4 + 2], l2 - lse)
        plsc.store_scatter(envout_v, [rl4 + 3], l3 - lse)
        return 0

    lax.fori_loop(0, NG, group1, 0)

    cps = (gather_rows(wue, idxu_v, rows_u)
           + gather_rows(wie, idxi_v, rows_i))
    for cp in cps:
        cp.wait()

    def group2(g, _):
        acc = jnp.zeros((L,), _f32)
        for r in range(L):
            row = g * L + r
            pt = [rows_u[row, pl.ds(t * L, L)] * rows_i[row, pl.ds(t * L, L)]
                  * rows_e[row, pl.ds(t * L, L)] for t in range(NT)]
            s = (pt[0] + pt[1]) + (pt[2] + pt[3])
            acc = lane_sum_into(acc, s, r)
        mid = _sigmoid(acc)
        envsc_v[pl.ds(g * L, L)] = invs_v[pl.ds(g * L, L)] * mid
        return 0

    lax.fori_loop(0, NG, group2, 0)

    pltpu.sync_copy(invs_v, o_inv.at[pl.ds(base, RPW)])
    pltpu.sync_copy(envsc_v, o_env.at[pl.ds(base, RPW)])
    pltpu.sync_copy(envout_v, o_cls.at[pl.ds(base * ENV, RPW * ENV)])


def kernel(users_id, items_id, envs_id, alpha, W_user_inv, W_item_inv,
           W_user_env, W_item_env, W_env, clf_W, clf_b):
    del alpha
    u2d = users_id.reshape(B // IDXC, IDXC)
    i2d = items_id.reshape(B // IDXC, IDXC)
    e2d = envs_id.reshape(B // IDXC, IDXC)
    clfb = jnp.zeros((L,), _f32).at[:ENV].set(clf_b)
    inv_s, env_s, env_out = _sc_forward(
        u2d, i2d, e2d, W_user_inv, W_item_inv, W_user_env, W_item_env,
        W_env, clf_W, clfb)
    return inv_s, env_s, env_out.reshape(B, ENV)
